# baseline (speedup 1.0000x reference)
import jax
import jax.numpy as jnp
from jax.experimental import pallas as pl
from jax.experimental.pallas import tpu as pltpu
from jax.experimental.pallas import tpu_sc as plsc

N_EXPERTS = 8
D_MODEL = 1024
N_TOKENS = 4096

BLK = 256
NBLK = N_TOKENS // BLK + 8
NPAD = NBLK * BLK
SPLIT = 4
D_SUB = D_MODEL // SPLIT
N_SUB = N_TOKENS * SPLIT
WIN = 128

_vector_mesh = plsc.VectorSubcoreMesh(core_axis_name="core",
                                      subcore_axis_name="subcore")


def _sc_scatter_rows(x, dst_sub_idx):
    x_sub = x.reshape(N_SUB, D_SUB)

    @pl.kernel(out_type=jax.ShapeDtypeStruct((NPAD * SPLIT, D_SUB), x.dtype),
               mesh=_vector_mesh)
    def scatter_kernel(x_hbm, i_hbm, o_hbm):
        def body(x_vmem, i_vmem):
            pltpu.sync_copy(x_vmem, o_hbm.at[i_vmem.at[0]])

        pltpu.emit_pipeline(
            body,
            grid=(N_SUB // WIN,),
            in_specs=[pl.BlockSpec((WIN, D_SUB), lambda i: (i, 0)),
                      pl.BlockSpec((1, WIN), lambda i: (0, i))],
            out_specs=[],
            core_axis_name=("core", "subcore"),
            dimension_semantics=(pltpu.PARALLEL,),
        )(x_hbm, i_hbm)

    return scatter_kernel(x_sub, dst_sub_idx).reshape(NPAD, D_MODEL)


def _sc_gather_rows(src, src_sub_idx):
    src_sub = src.reshape(NPAD * SPLIT, D_SUB)

    @pl.kernel(out_type=jax.ShapeDtypeStruct((N_SUB, D_SUB), src.dtype),
               mesh=_vector_mesh)
    def gather_kernel(src_hbm, i_hbm, o_hbm):
        def body(i_vmem, o_vmem):
            pltpu.sync_copy(src_hbm.at[i_vmem.at[0]], o_vmem)

        pltpu.emit_pipeline(
            body,
            grid=(N_SUB // WIN,),
            in_specs=[pl.BlockSpec((1, WIN), lambda i: (0, i))],
            out_specs=[pl.BlockSpec((WIN, D_SUB), lambda i: (i, 0))],
            core_axis_name=("core", "subcore"),
            dimension_semantics=(pltpu.PARALLEL,),
        )(i_hbm, o_hbm)

    return gather_kernel(src_sub, src_sub_idx).reshape(N_TOKENS, D_MODEL)


def _tc_expert_matmul(x_pad, W, b, expert_of_block, nblk_total):
    def mm_kernel(e_ref, v_ref, x_ref, w_ref, b_ref, o_ref):
        @pl.when(pl.program_id(0) < v_ref[0])
        def _():
            o_ref[...] = (jnp.dot(x_ref[...].astype(jnp.bfloat16),
                                  w_ref[0].astype(jnp.bfloat16),
                                  preferred_element_type=jnp.float32)
                          + b_ref[0])

    grid_spec = pltpu.PrefetchScalarGridSpec(
        num_scalar_prefetch=2,
        grid=(NBLK,),
        in_specs=[
            pl.BlockSpec((BLK, D_MODEL),
                         lambda i, e, v: (jnp.minimum(i, v[0] - 1), 0)),
            pl.BlockSpec((1, D_MODEL, D_MODEL), lambda i, e, v: (e[i], 0, 0)),
            pl.BlockSpec((1, 1, D_MODEL), lambda i, e, v: (e[i], 0, 0)),
        ],
        out_specs=pl.BlockSpec((BLK, D_MODEL),
                               lambda i, e, v: (jnp.minimum(i, v[0] - 1), 0)),
    )
    return pl.pallas_call(
        mm_kernel,
        grid_spec=grid_spec,
        out_shape=jax.ShapeDtypeStruct((NPAD, D_MODEL), jnp.float32),
    )(expert_of_block, nblk_total, x_pad, W, b.reshape(N_EXPERTS, 1, D_MODEL))


def kernel(x, groups, W, b):
    g = groups[:, 0].astype(jnp.int32)

    onehot = (g[:, None] == jnp.arange(N_EXPERTS, dtype=jnp.int32)[None, :])
    onehot_i = onehot.astype(jnp.int32)
    cum = jnp.cumsum(onehot_i, axis=0)
    counts = cum[-1]
    nblk = (counts + BLK - 1) // BLK
    cum_nblk = jnp.cumsum(nblk)
    nblk_total = cum_nblk[-1:].astype(jnp.int32)
    pad_start = (cum_nblk - nblk) * BLK
    padpos = jnp.sum(onehot_i * (pad_start[None, :] + cum - 1),
                     axis=1).astype(jnp.int32)
    sub_idx = (padpos[:, None] * SPLIT
               + jnp.arange(SPLIT, dtype=jnp.int32)[None, :]).reshape(1, N_SUB)
    expert_of_block = jnp.minimum(
        jnp.searchsorted(cum_nblk, jnp.arange(NBLK), side="right"),
        N_EXPERTS - 1).astype(jnp.int32)

    x_pad = _sc_scatter_rows(x, sub_idx)
    out_pad = _tc_expert_matmul(x_pad, W, b, expert_of_block, nblk_total)
    return _sc_gather_rows(out_pad, sub_idx)


# speedup vs baseline: 1.6641x; 1.6641x over previous
import functools

import jax
import jax.numpy as jnp
from jax import lax
from jax.experimental import pallas as pl
from jax.experimental.pallas import tpu as pltpu
from jax.experimental.pallas import tpu_sc as plsc

N_EXPERTS = 8
D_MODEL = 1024
N_TOKENS = 4096

BLK = 256
NBLK = N_TOKENS // BLK + 8
NPAD = NBLK * BLK

NW = 32
TPW = N_TOKENS // NW
GCH = 64
NCH = TPW // GCH

_mesh = plsc.VectorSubcoreMesh(core_axis_name="core", subcore_axis_name="subcore")


@functools.partial(
    pl.kernel,
    out_type=jax.ShapeDtypeStruct((NPAD, D_MODEL), jnp.float32),
    mesh=_mesh,
    scratch_types=[
        pltpu.VMEM((NCH, GCH), jnp.int32),
        pltpu.VMEM((GCH, D_MODEL), jnp.float32),
    ],
)
def _sc_scatter(x_hbm, idx_hbm, o_hbm, idx_v, rows_v):
    wid = lax.axis_index("subcore") * 2 + lax.axis_index("core")
    base = wid * TPW
    for c in range(NCH):
        pltpu.sync_copy(idx_hbm.at[pl.ds(base + c * GCH, GCH)], idx_v.at[c])
        pltpu.sync_copy(x_hbm.at[pl.ds(base + c * GCH, GCH)], rows_v)
        pltpu.sync_copy(rows_v, o_hbm.at[idx_v.at[c]])


@functools.partial(
    pl.kernel,
    out_type=jax.ShapeDtypeStruct((N_TOKENS, D_MODEL), jnp.float32),
    mesh=_mesh,
    scratch_types=[
        pltpu.VMEM((NCH, GCH), jnp.int32),
        pltpu.VMEM((GCH, D_MODEL), jnp.float32),
    ],
)
def _sc_gather(src_hbm, idx_hbm, o_hbm, idx_v, rows_v):
    wid = lax.axis_index("subcore") * 2 + lax.axis_index("core")
    base = wid * TPW
    for c in range(NCH):
        pltpu.sync_copy(idx_hbm.at[pl.ds(base + c * GCH, GCH)], idx_v.at[c])
        pltpu.sync_copy(src_hbm.at[idx_v.at[c]], rows_v)
        pltpu.sync_copy(rows_v, o_hbm.at[pl.ds(base + c * GCH, GCH)])


def _tc_expert_matmul(x_pad, W, b, expert_of_block, nblk_total):
    def mm_kernel(e_ref, v_ref, x_ref, w_ref, b_ref, o_ref):
        @pl.when(pl.program_id(0) < v_ref[0])
        def _():
            o_ref[...] = (jnp.dot(x_ref[...], w_ref[0],
                                  preferred_element_type=jnp.float32)
                          + b_ref[0])

    grid_spec = pltpu.PrefetchScalarGridSpec(
        num_scalar_prefetch=2,
        grid=(NBLK,),
        in_specs=[
            pl.BlockSpec((BLK, D_MODEL),
                         lambda i, e, v: (jnp.minimum(i, v[0] - 1), 0)),
            pl.BlockSpec((1, D_MODEL, D_MODEL), lambda i, e, v: (e[i], 0, 0)),
            pl.BlockSpec((1, 1, D_MODEL), lambda i, e, v: (e[i], 0, 0)),
        ],
        out_specs=pl.BlockSpec((BLK, D_MODEL),
                               lambda i, e, v: (jnp.minimum(i, v[0] - 1), 0)),
    )
    return pl.pallas_call(
        mm_kernel,
        grid_spec=grid_spec,
        out_shape=jax.ShapeDtypeStruct((NPAD, D_MODEL), jnp.float32),
    )(expert_of_block, nblk_total, x_pad, W, b.reshape(N_EXPERTS, 1, D_MODEL))


def kernel(x, groups, W, b):
    g = groups[:, 0].astype(jnp.int32)

    onehot_i = (g[:, None] == jnp.arange(N_EXPERTS, dtype=jnp.int32)[None, :]
                ).astype(jnp.int32)
    cum = jnp.cumsum(onehot_i, axis=0)
    counts = cum[-1]
    nblk = (counts + BLK - 1) // BLK
    cum_nblk = jnp.cumsum(nblk)
    nblk_total = cum_nblk[-1:].astype(jnp.int32)
    pad_start = (cum_nblk - nblk) * BLK
    padpos = jnp.sum(onehot_i * (pad_start[None, :] + cum - 1),
                     axis=1).astype(jnp.int32)
    expert_of_block = jnp.minimum(
        jnp.searchsorted(cum_nblk, jnp.arange(NBLK), side="right"),
        N_EXPERTS - 1).astype(jnp.int32)

    Wb = W.astype(jnp.bfloat16)
    x_pad = _sc_scatter(x, padpos)
    xb_pad = x_pad.astype(jnp.bfloat16)
    out_pad = _tc_expert_matmul(xb_pad, Wb, b, expert_of_block, nblk_total)
    return _sc_gather(out_pad, padpos)


# speedup vs baseline: 1.9005x; 1.1421x over previous
import functools

import jax
import jax.numpy as jnp
from jax import lax
from jax.experimental import pallas as pl
from jax.experimental.pallas import tpu as pltpu
from jax.experimental.pallas import tpu_sc as plsc

N_EXPERTS = 8
D_MODEL = 1024
N_TOKENS = 4096

BLK = 256
NBLK = N_TOKENS // BLK + 8
NPAD = NBLK * BLK

NW = 32
TPW = N_TOKENS // NW
GCH = 64
NCH = TPW // GCH

_mesh = plsc.VectorSubcoreMesh(core_axis_name="core", subcore_axis_name="subcore")


@functools.partial(
    pl.kernel,
    out_type=jax.ShapeDtypeStruct((NPAD, D_MODEL), jnp.float32),
    mesh=_mesh,
    scratch_types=[
        pltpu.VMEM((NCH, GCH), jnp.int32),
        pltpu.VMEM((GCH, D_MODEL), jnp.float32),
    ],
)
def _sc_scatter(x_hbm, idx_hbm, o_hbm, idx_v, rows_v):
    wid = lax.axis_index("subcore") * 2 + lax.axis_index("core")
    base = wid * TPW
    for c in range(NCH):
        pltpu.sync_copy(idx_hbm.at[pl.ds(base + c * GCH, GCH)], idx_v.at[c])
        pltpu.sync_copy(x_hbm.at[pl.ds(base + c * GCH, GCH)], rows_v)
        pltpu.sync_copy(rows_v, o_hbm.at[idx_v.at[c]])


@functools.partial(
    pl.kernel,
    out_type=jax.ShapeDtypeStruct((N_TOKENS, D_MODEL), jnp.float32),
    mesh=_mesh,
    scratch_types=[
        pltpu.VMEM((NCH, GCH), jnp.int32),
        pltpu.VMEM((GCH, D_MODEL), jnp.float32),
    ],
)
def _sc_gather(src_hbm, idx_hbm, o_hbm, idx_v, rows_v):
    wid = lax.axis_index("subcore") * 2 + lax.axis_index("core")
    base = wid * TPW
    for c in range(NCH):
        pltpu.sync_copy(idx_hbm.at[pl.ds(base + c * GCH, GCH)], idx_v.at[c])
        pltpu.sync_copy(src_hbm.at[idx_v.at[c]], rows_v)
        pltpu.sync_copy(rows_v, o_hbm.at[pl.ds(base + c * GCH, GCH)])


def _tc_expert_matmul(x_pad, W, b, expert_of_block, nblk_total):
    def mm_kernel(e_ref, v_ref, x_ref, w_ref, b_ref, o_ref):
        @pl.when(pl.program_id(0) < v_ref[0])
        def _():
            o_ref[...] = (jnp.dot(x_ref[...].astype(jnp.bfloat16), w_ref[0],
                                  preferred_element_type=jnp.float32)
                          + b_ref[0])

    grid_spec = pltpu.PrefetchScalarGridSpec(
        num_scalar_prefetch=2,
        grid=(NBLK,),
        in_specs=[
            pl.BlockSpec((BLK, D_MODEL),
                         lambda i, e, v: (jnp.minimum(i, v[0] - 1), 0)),
            pl.BlockSpec((1, D_MODEL, D_MODEL), lambda i, e, v: (e[i], 0, 0)),
            pl.BlockSpec((1, 1, D_MODEL), lambda i, e, v: (e[i], 0, 0)),
        ],
        out_specs=pl.BlockSpec((BLK, D_MODEL),
                               lambda i, e, v: (jnp.minimum(i, v[0] - 1), 0)),
    )
    return pl.pallas_call(
        mm_kernel,
        grid_spec=grid_spec,
        out_shape=jax.ShapeDtypeStruct((NPAD, D_MODEL), jnp.float32),
    )(expert_of_block, nblk_total, x_pad, W, b.reshape(N_EXPERTS, 1, D_MODEL))


def kernel(x, groups, W, b):
    g = groups[:, 0].astype(jnp.int32)

    onehot_i = (g[:, None] == jnp.arange(N_EXPERTS, dtype=jnp.int32)[None, :]
                ).astype(jnp.int32)
    cum = jnp.cumsum(onehot_i, axis=0)
    counts = cum[-1]
    nblk = (counts + BLK - 1) // BLK
    cum_nblk = jnp.cumsum(nblk)
    nblk_total = cum_nblk[-1:].astype(jnp.int32)
    pad_start = (cum_nblk - nblk) * BLK
    padpos = jnp.sum(onehot_i * (pad_start[None, :] + cum - 1),
                     axis=1).astype(jnp.int32)
    expert_of_block = jnp.minimum(
        jnp.sum((jnp.arange(NBLK, dtype=jnp.int32)[:, None]
                 >= cum_nblk[None, :]).astype(jnp.int32), axis=1),
        N_EXPERTS - 1).astype(jnp.int32)

    Wb = W.astype(jnp.bfloat16)
    x_pad = _sc_scatter(x, padpos)
    out_pad = _tc_expert_matmul(x_pad, Wb, b, expert_of_block, nblk_total)
    return _sc_gather(out_pad, padpos)
